# baseline (speedup 1.0000x reference)
import functools

import jax
import jax.numpy as jnp
from jax import lax
from jax.experimental import pallas as pl
from jax.experimental.pallas import tpu as pltpu
from jax.experimental.pallas import tpu_sc as plsc

_DIM = 64
_NC, _NS, _L = 2, 16, 16
_NW = _NC * _NS
_LANE_ROW = 128


def _build(n_rows, seq, chunk_rows):
    r_per_w = n_rows // (_NW * _LANE_ROW)
    assert r_per_w % chunk_rows == 0
    nchunks = r_per_w // chunk_rows
    chunk = chunk_rows * _LANE_ROW

    mesh = plsc.VectorSubcoreMesh(core_axis_name="c", subcore_axis_name="s")

    @functools.partial(
        pl.kernel,
        out_type=jax.ShapeDtypeStruct((n_rows, _DIM), jnp.float32),
        mesh=mesh,
        compiler_params=pltpu.CompilerParams(use_tc_tiling_on_sc=False),
        scratch_types=[
            pltpu.VMEM((chunk_rows, _LANE_ROW), jnp.int32),
            pltpu.VMEM((chunk, _DIM), jnp.float32),
            pltpu.VMEM((_L,), jnp.int32),
            pltpu.SemaphoreType.DMA,
        ],
    )
    def emb(text_hbm, seqlen_hbm, table_hbm, out_hbm, idx_v, rows_v, sl_v, gsem):
        wid = lax.axis_index("s") * _NC + lax.axis_index("c")
        rbase = wid * r_per_w
        pltpu.sync_copy(seqlen_hbm, sl_v)
        sl = sl_v[...]
        lanes = lax.iota(jnp.int32, _L)

        def chunk_body(c, carry):
            crow = rbase + c * chunk_rows
            pltpu.sync_copy(text_hbm.at[pl.ds(crow, chunk_rows)], idx_v)
            for k in range(chunk_rows):
                row_ref = idx_v.at[k]
                for j in range(_LANE_ROW // _L):
                    v = row_ref[pl.ds(j * _L, _L)]
                    p = (crow + k) * _LANE_ROW + j * _L + lanes
                    s = lax.rem(p, jnp.int32(seq))
                    row_ref[pl.ds(j * _L, _L)] = jnp.where(s < sl, v + 1, 0)
            copies = [
                pltpu.async_copy(
                    table_hbm.at[idx_v.at[k]],
                    rows_v.at[pl.ds(k * _LANE_ROW, _LANE_ROW)],
                    gsem,
                )
                for k in range(chunk_rows)
            ]
            for cp in copies:
                cp.wait()
            pltpu.sync_copy(rows_v, out_hbm.at[pl.ds(crow * _LANE_ROW, chunk)])
            return carry

        lax.fori_loop(0, nchunks, chunk_body, 0)

    return emb


@functools.lru_cache(maxsize=None)
def _cached(n_rows, seq, chunk_rows):
    return _build(n_rows, seq, chunk_rows)


def kernel(text, seq_len, table):
    b, s = text.shape
    n = b * s
    text2d = text.astype(jnp.int32).reshape(n // _LANE_ROW, _LANE_ROW)
    slv = jnp.full((_L,), seq_len, dtype=jnp.int32)
    out = _cached(n, s, chunk_rows=4)(text2d, slv, table)
    return out.reshape(b, s, _DIM)
